# speedup vs baseline: 13.9675x; 13.9675x over previous
import functools

import jax
import jax.numpy as jnp
from jax import lax
from jax.experimental import pallas as pl
from jax.experimental.pallas import tpu as pltpu
from jax.experimental.pallas import tpu_sc as plsc

N = 10000
DF = 128
NCLS = 40
E = 320000

NC = 2
NS = 16
NW = NC * NS
K = 128
SUB = 79
CT = SUB * K
EP = NW * CT
NP = 10240
RPT = NP // NS

_mesh = functools.partial(
    plsc.VectorSubcoreMesh,
    core_axis_name="c",
    subcore_axis_name="s",
    num_cores=NC,
    num_subcores=NS,
)


def _deg_body(dst_hbm, zeros_hbm, out_hbm, dst_v, ones_v, acc_sp):
    c = lax.axis_index("c")
    s = lax.axis_index("s")
    w = c * NS + s
    pltpu.sync_copy(dst_hbm.at[w], dst_v)
    ones16 = jnp.ones((16,), jnp.float32)
    for i in range(K // 16):
        ones_v[pl.ds(i * 16, 16)] = ones16
    pltpu.sync_copy(zeros_hbm.at[pl.ds(s * RPT, RPT)], acc_sp.at[pl.ds(s * RPT, RPT)])
    plsc.subcore_barrier()

    def body(j, carry):
        pltpu.sync_copy(ones_v, acc_sp.at[dst_v.at[j]], add=True)
        return carry

    lax.fori_loop(0, SUB, body, 0)
    plsc.subcore_barrier()
    pltpu.sync_copy(acc_sp.at[pl.ds(s * RPT, RPT)], out_hbm.at[c, pl.ds(s * RPT, RPT)])


def _edge_pass_body(table_hbm, src_hbm, dst_hbm, zeros_hbm, out_hbm,
                    src_v, dst_v, rows_v, acc_sp):
    c = lax.axis_index("c")
    s = lax.axis_index("s")
    w = c * NS + s
    pltpu.sync_copy(src_hbm.at[w], src_v)
    pltpu.sync_copy(dst_hbm.at[w], dst_v)
    pltpu.sync_copy(zeros_hbm, acc_sp.at[pl.ds(s * RPT, RPT)])
    plsc.subcore_barrier()

    def body(j, carry):
        pltpu.sync_copy(table_hbm.at[src_v.at[j]], rows_v)
        pltpu.sync_copy(rows_v, acc_sp.at[dst_v.at[j]], add=True)
        return carry

    lax.fori_loop(0, SUB, body, 0)
    plsc.subcore_barrier()
    pltpu.sync_copy(acc_sp.at[pl.ds(s * RPT, RPT)],
                    out_hbm.at[c, pl.ds(s * RPT, RPT)])


_deg_kernel = pl.kernel(
    _deg_body,
    out_type=jax.ShapeDtypeStruct((NC, NP), jnp.float32),
    mesh=_mesh(),
    scratch_types=[
        pltpu.VMEM((SUB, K), jnp.int32),
        pltpu.VMEM((K,), jnp.float32),
        pltpu.VMEM_SHARED((NP,), jnp.float32),
    ],
)

_edge_kernel = pl.kernel(
    _edge_pass_body,
    out_type=jax.ShapeDtypeStruct((NC, NP, DF), jnp.float32),
    mesh=_mesh(),
    scratch_types=[
        pltpu.VMEM((SUB, K), jnp.int32),
        pltpu.VMEM((SUB, K), jnp.int32),
        pltpu.VMEM((K, DF), jnp.float32),
        pltpu.VMEM_SHARED((NP, DF), jnp.float32),
    ],
)


_R = 1000


def _tc1_body(x_ref, degp_ref, wpre_ref, bpre_ref, w1_ref, g1p_ref, dinv_ref):
    deg = degp_ref[0] + degp_ref[1] + 1.0
    dinv = lax.rsqrt(deg)
    h0 = jnp.dot(x_ref[...], wpre_ref[...], preferred_element_type=jnp.float32)
    h0 = h0 + bpre_ref[...]
    g1 = jnp.dot(h0, w1_ref[...], preferred_element_type=jnp.float32)
    g1p_ref[...] = g1 * dinv
    dinv_ref[...] = dinv


def _tc2_body(acc_ref, g1p_ref, dinv_ref, b1_ref, w2_ref, g2p_ref):
    dinv = dinv_ref[...]
    agg = dinv * (acc_ref[0] + acc_ref[1] + g1p_ref[...]) + b1_ref[...]
    h1 = jnp.maximum(agg, 0.0)
    g2p_ref[...] = jnp.dot(h1, w2_ref[...], preferred_element_type=jnp.float32) * dinv


def _tc3_body(acc_ref, g2p_ref, dinv_ref, b2_ref, wcls_ref, bcls_ref, out_ref):
    dinv = dinv_ref[...]
    h2 = dinv * (acc_ref[0] + acc_ref[1] + g2p_ref[...]) + b2_ref[...]
    nrm = jnp.sqrt(jnp.sum(h2 * h2, axis=-1, keepdims=True))
    h = h2 / jnp.maximum(nrm, 1e-12)
    logits = jnp.dot(h, wcls_ref[...], preferred_element_type=jnp.float32)
    logits = logits + bcls_ref[...]
    m = jnp.max(logits, axis=-1, keepdims=True)
    lse = m + jnp.log(jnp.sum(jnp.exp(logits - m), axis=-1, keepdims=True))
    out_ref[...] = logits - lse


def _row_spec(shape):
    if len(shape) == 2:
        return pl.BlockSpec((_R, shape[1]), lambda i: (i, 0))
    return pl.BlockSpec((shape[0], _R, shape[2]), lambda i: (0, i, 0))


def _full_spec(shape):
    nd = len(shape)
    return pl.BlockSpec(shape, lambda i: (0,) * nd)


def _tc1(x, degp, wpre, bpre, w1):
    return pl.pallas_call(
        _tc1_body,
        grid=(N // _R,),
        in_specs=[
            _row_spec((N, DF)),
            _row_spec((2, N, 1)),
            _full_spec((DF, DF)),
            _full_spec((1, DF)),
            _full_spec((DF, DF)),
        ],
        out_specs=[_row_spec((N, DF)), _row_spec((N, 1))],
        out_shape=[
            jax.ShapeDtypeStruct((N, DF), jnp.float32),
            jax.ShapeDtypeStruct((N, 1), jnp.float32),
        ],
    )(x, degp, wpre, bpre, w1)


def _tc2(acc, g1p, dinv, b1, w2):
    return pl.pallas_call(
        _tc2_body,
        grid=(N // _R,),
        in_specs=[
            _row_spec((2, N, DF)),
            _row_spec((N, DF)),
            _row_spec((N, 1)),
            _full_spec((1, DF)),
            _full_spec((DF, DF)),
        ],
        out_specs=_row_spec((N, DF)),
        out_shape=jax.ShapeDtypeStruct((N, DF), jnp.float32),
    )(acc, g1p, dinv, b1, w2)


def _tc3(acc, g2p, dinv, b2, wcls, bcls):
    return pl.pallas_call(
        _tc3_body,
        grid=(N // _R,),
        in_specs=[
            _row_spec((2, N, DF)),
            _row_spec((N, DF)),
            _row_spec((N, 1)),
            _full_spec((1, DF)),
            _full_spec((DF, NCLS)),
            _full_spec((1, NCLS)),
        ],
        out_specs=pl.BlockSpec((_R, NCLS), lambda i: (i, 0)),
        out_shape=jax.ShapeDtypeStruct((N, NCLS), jnp.float32),
    )(acc, g2p, dinv, b2, wcls, bcls)


def kernel(x, edge_index, W_pre, b_pre, W1, b1, W2, b2, W_cls, b_cls):
    src = edge_index[0]
    dst = edge_index[1]
    pad = EP - E
    srcp = jnp.concatenate([src, jnp.zeros((pad,), jnp.int32)]).reshape(NW, SUB, K)
    dstp = jnp.concatenate([dst, jnp.full((pad,), N, jnp.int32)]).reshape(NW, SUB, K)

    zflat = jnp.zeros((NP,), jnp.float32)
    zrows = jnp.zeros((RPT, DF), jnp.float32)

    degp = _deg_kernel(dstp, zflat)
    degp_sl = degp[:, :N, None]

    g1p, dinv = _tc1(x, degp_sl, W_pre, b_pre.reshape(1, DF), W1)

    acc1 = _edge_kernel(g1p, srcp, dstp, zrows)
    g2p = _tc2(acc1[:, :N, :], g1p, dinv, b1.reshape(1, DF), W2)

    acc2 = _edge_kernel(g2p, srcp, dstp, zrows)
    return _tc3(acc2[:, :N, :], g2p, dinv, b2.reshape(1, DF),
                W_cls, b_cls.reshape(1, NCLS))
